# baseline (speedup 1.0000x reference)
import jax
import jax.numpy as jnp
import numpy as np
from jax import lax
from jax.experimental import pallas as pl
from jax.experimental.pallas import tpu as pltpu

_ROWS = 8
_W = 1024
_THREEFRY_C = 0x1BD11BDA
_F32_ONE_BITS = 0x3F800000
_TINY = np.float32(np.finfo(np.float32).tiny)
_NEG_INF = np.float32(-np.inf)
_INT_MAX = np.int32(2**31 - 1)


def _rotl(x, r):
    return (x << jnp.int32(r)) | lax.shift_right_logical(x, jnp.int32(32 - r))


def _threefry2x32(k0, k1, c1):
    k2 = k0 ^ k1 ^ jnp.int32(_THREEFRY_C)
    x0 = k0
    x1 = c1 + k1
    ks = (k0, k1, k2)
    rots = ((13, 15, 26, 6), (17, 29, 16, 24),
            (13, 15, 26, 6), (17, 29, 16, 24), (13, 15, 26, 6))
    for d in range(5):
        for r in rots[d]:
            x0 = x0 + x1
            x1 = _rotl(x1, r) ^ x0
        x0 = x0 + ks[(d + 1) % 3]
        x1 = x1 + ks[(d + 2) % 3] + jnp.int32(d + 1)
    return x0 ^ x1


def _bits_to_unit_float(bits):
    fb = lax.shift_right_logical(bits, jnp.int32(9)) | jnp.int32(_F32_ONE_BITS)
    return lax.bitcast_convert_type(fb, jnp.float32) - jnp.float32(1.0)


def _make_body(V):
    def _body(scal_ref, logits_ref, target_ref, out_ref, acc_s_ref, acc_c_ref):
        i = pl.program_id(0)
        j = pl.program_id(1)
        nj = pl.num_programs(1)
        key0 = scal_ref[0]
        key1 = scal_ref[1]

        x = logits_ref[...]
        R, W = x.shape

        rowoff = (lax.broadcasted_iota(jnp.int32, (R, 1), 0) + i * R) * V
        col = lax.broadcasted_iota(jnp.int32, (R, W), 1) + j * W
        bits = _threefry2x32(key0, key1, rowoff + col)
        floats = _bits_to_unit_float(bits)
        u = jnp.maximum(_TINY, floats * (jnp.float32(1.0) - _TINY) + _TINY)
        g = -jnp.log(-jnp.log(u))

        score = jnp.where(col < V, x + g, _NEG_INF)

        @pl.when(j == 0)
        def _init():
            acc_s_ref[...] = jnp.full((R, W), _NEG_INF, jnp.float32)
            acc_c_ref[...] = jnp.full((R, W), _INT_MAX, jnp.int32)

        acc_s = acc_s_ref[...]
        take = score > acc_s
        acc_s_ref[...] = jnp.where(take, score, acc_s)
        acc_c_ref[...] = jnp.where(take, col, acc_c_ref[...])

        @pl.when(j == nj - 1)
        def _finish():
            a_s = acc_s_ref[...]
            a_c = acc_c_ref[...]
            best = jnp.max(a_s, axis=1, keepdims=True)
            idx = jnp.min(jnp.where(a_s == best, a_c, _INT_MAX),
                          axis=1, keepdims=True)
            sample = idx.astype(jnp.float32)

            rctr = lax.broadcasted_iota(jnp.int32, (R, 1), 0) + i * R
            cbits = _threefry2x32(scal_ref[2], scal_ref[3], rctr)
            cp = jnp.maximum(jnp.float32(0.0), _bits_to_unit_float(cbits))

            t = target_ref[...]
            tcol = lax.broadcasted_iota(jnp.int32, t.shape, 1)
            tgt = jnp.sum(jnp.where(tcol == scal_ref[4], t, jnp.float32(0.0)),
                          axis=1, keepdims=True)

            thr = lax.bitcast_convert_type(scal_ref[5], jnp.float32)
            out_ref[...] = jnp.where(cp < thr, tgt, sample)

    return _body


def kernel(target, logits, step, summary_step):
    B, V = logits.shape
    T = target.shape[1]

    skd = lax.bitcast_convert_type(
        jax.random.key_data(jax.random.fold_in(jax.random.key(42), summary_step)),
        jnp.int32)
    ckd = lax.bitcast_convert_type(
        jax.random.key_data(jax.random.fold_in(jax.random.key(7), step)),
        jnp.int32)
    stepf = jnp.asarray(step, jnp.float32)
    thr = jnp.float32(100.0) / (jnp.float32(100.0) + jnp.exp(stepf / jnp.float32(100.0)))
    scalars = jnp.concatenate([
        skd.reshape(2), ckd.reshape(2),
        jnp.asarray(step, jnp.int32).reshape(1),
        lax.bitcast_convert_type(thr, jnp.int32).reshape(1),
    ])

    grid = (B // _ROWS, pl.cdiv(V, _W))
    out = pl.pallas_call(
        _make_body(V),
        grid=grid,
        in_specs=[
            pl.BlockSpec(memory_space=pltpu.SMEM),
            pl.BlockSpec((_ROWS, _W), lambda i, j: (i, j)),
            pl.BlockSpec((_ROWS, T), lambda i, j: (i, 0)),
        ],
        out_specs=pl.BlockSpec((_ROWS, 1), lambda i, j: (i, 0)),
        out_shape=jax.ShapeDtypeStruct((B, 1), jnp.float32),
        scratch_shapes=[
            pltpu.VMEM((_ROWS, _W), jnp.float32),
            pltpu.VMEM((_ROWS, _W), jnp.int32),
        ],
    )(scalars, logits, target)
    return out


# speedup vs baseline: 1.5729x; 1.5729x over previous
import jax
import jax.numpy as jnp
import numpy as np
from jax import lax
from jax.experimental import pallas as pl
from jax.experimental.pallas import tpu as pltpu

_ROWS = 8
_W = 2048
_THREEFRY_C = 0x1BD11BDA
_F32_ONE_BITS = 0x3F800000
_TINY = np.float32(np.finfo(np.float32).tiny)
_NEG_INF = np.float32(-np.inf)
_INT_MAX = np.int32(2**31 - 1)


def _rotl(x, r):
    return (x << jnp.int32(r)) | lax.shift_right_logical(x, jnp.int32(32 - r))


def _threefry2x32(k0, k1, c1):
    k2 = k0 ^ k1 ^ jnp.int32(_THREEFRY_C)
    x0 = k0
    x1 = c1 + k1
    ks = (k0, k1, k2)
    rots = ((13, 15, 26, 6), (17, 29, 16, 24),
            (13, 15, 26, 6), (17, 29, 16, 24), (13, 15, 26, 6))
    for d in range(5):
        for r in rots[d]:
            x0 = x0 + x1
            x1 = _rotl(x1, r) ^ x0
        x0 = x0 + ks[(d + 1) % 3]
        x1 = x1 + ks[(d + 2) % 3] + jnp.int32(d + 1)
    return x0 ^ x1


def _bits_to_unit_float(bits):
    fb = lax.shift_right_logical(bits, jnp.int32(9)) | jnp.int32(_F32_ONE_BITS)
    return lax.bitcast_convert_type(fb, jnp.float32) - jnp.float32(1.0)


def _make_scan_body(V):
    def _body(scal_ref, logits_ref, acc_s_ref, acc_c_ref):
        i = pl.program_id(0)
        j = pl.program_id(1)

        x = logits_ref[...]
        R, W = x.shape

        rowoff = (lax.broadcasted_iota(jnp.int32, (R, 1), 0) + i * R) * V
        col = lax.broadcasted_iota(jnp.int32, (R, W), 1) + j * W
        bits = _threefry2x32(scal_ref[0], scal_ref[1], rowoff + col)
        floats = _bits_to_unit_float(bits)
        u = jnp.maximum(_TINY, floats * (jnp.float32(1.0) - _TINY) + _TINY)
        g = -jnp.log(-jnp.log(u))

        score = jnp.where(col < V, x + g, _NEG_INF)

        take = jnp.logical_or(score > acc_s_ref[...], j == 0)
        acc_s_ref[...] = jnp.where(take, score, acc_s_ref[...])
        acc_c_ref[...] = jnp.where(take, col, acc_c_ref[...])

    return _body


def _finish_body(scal_ref, acc_s_ref, acc_c_ref, target_ref, out_ref):
    i = pl.program_id(0)
    a_s = acc_s_ref[...]
    a_c = acc_c_ref[...]
    R = a_s.shape[0]

    best = jnp.max(a_s, axis=1, keepdims=True)
    idx = jnp.min(jnp.where(a_s == best, a_c, _INT_MAX), axis=1, keepdims=True)
    sample = idx.astype(jnp.float32)

    rctr = lax.broadcasted_iota(jnp.int32, (R, 1), 0) + i * R
    cbits = _threefry2x32(scal_ref[2], scal_ref[3], rctr)
    cp = jnp.maximum(jnp.float32(0.0), _bits_to_unit_float(cbits))

    t = target_ref[...]
    tcol = lax.broadcasted_iota(jnp.int32, t.shape, 1)
    tgt = jnp.sum(jnp.where(tcol == scal_ref[4], t, jnp.float32(0.0)),
                  axis=1, keepdims=True)

    thr = lax.bitcast_convert_type(scal_ref[5], jnp.float32)
    out_ref[...] = jnp.where(cp < thr, tgt, sample)


def kernel(target, logits, step, summary_step):
    B, V = logits.shape
    T = target.shape[1]

    skd = lax.bitcast_convert_type(
        jax.random.key_data(jax.random.fold_in(jax.random.key(42), summary_step)),
        jnp.int32)
    ckd = lax.bitcast_convert_type(
        jax.random.key_data(jax.random.fold_in(jax.random.key(7), step)),
        jnp.int32)
    stepf = jnp.asarray(step, jnp.float32)
    thr = jnp.float32(100.0) / (jnp.float32(100.0) + jnp.exp(stepf / jnp.float32(100.0)))
    scalars = jnp.concatenate([
        skd.reshape(2), ckd.reshape(2),
        jnp.asarray(step, jnp.int32).reshape(1),
        lax.bitcast_convert_type(thr, jnp.int32).reshape(1),
    ])

    acc_s, acc_c = pl.pallas_call(
        _make_scan_body(V),
        grid=(B // _ROWS, pl.cdiv(V, _W)),
        in_specs=[
            pl.BlockSpec(memory_space=pltpu.SMEM),
            pl.BlockSpec((_ROWS, _W), lambda i, j: (i, j)),
        ],
        out_specs=[
            pl.BlockSpec((_ROWS, _W), lambda i, j: (i, 0)),
            pl.BlockSpec((_ROWS, _W), lambda i, j: (i, 0)),
        ],
        out_shape=[
            jax.ShapeDtypeStruct((B, _W), jnp.float32),
            jax.ShapeDtypeStruct((B, _W), jnp.int32),
        ],
    )(scalars, logits)

    out = pl.pallas_call(
        _finish_body,
        grid=(B // _ROWS,),
        in_specs=[
            pl.BlockSpec(memory_space=pltpu.SMEM),
            pl.BlockSpec((_ROWS, _W), lambda i: (i, 0)),
            pl.BlockSpec((_ROWS, _W), lambda i: (i, 0)),
            pl.BlockSpec((_ROWS, T), lambda i: (i, 0)),
        ],
        out_specs=pl.BlockSpec((_ROWS, 1), lambda i: (i, 0)),
        out_shape=jax.ShapeDtypeStruct((B, 1), jnp.float32),
    )(scalars, acc_s, acc_c, target)
    return out
